# baseline (speedup 1.0000x reference)
import functools

import jax
import jax.numpy as jnp
from jax import lax
from jax.experimental import pallas as pl
from jax.experimental.pallas import tpu as pltpu
from jax.experimental.pallas import tpu_sc as plsc

IN_C = 128
OUT_C = 128
HID = 256
NX = 8
NY = 8
NB = 4
NSEG = NB * NX * NY
N = 50000
NPAD = 50176
BLK = 512
NGRID = NPAD // BLK
NC = 2
NS = 16
NW = NC * NS
RPW = NPAD // NW
SB = 112
NBLK = RPW // SB
UW = 384


NREP = NW


def _src_pool_body(idx_ref, x_ref, w1_ref, b1_ref, w2_ref, b2_ref, w3p_ref,
                   tblp_ref, sums_ref):
    i = pl.program_id(0)
    u = jnp.dot(x_ref[...], w1_ref[...], preferred_element_type=jnp.float32)
    u = jax.nn.gelu(u + b1_ref[...])
    u_ext = jnp.concatenate(
        [u, jnp.ones((BLK, UW - HID), jnp.float32)], axis=1)
    rid = i * BLK + jax.lax.broadcasted_iota(jnp.int32, (BLK, 1), 0)
    u_ext = jnp.where(rid < N, u_ext, 0.0)
    seg = idx_ref[0]
    oht = (seg == jax.lax.broadcasted_iota(jnp.int32, (NSEG, BLK), 0)
           ).astype(jnp.float32)
    contrib = jnp.dot(oht, u_ext,
                      preferred_element_type=jnp.float32)

    @pl.when(i == 0)
    def _():
        sums_ref[...] = contrib

    @pl.when(i > 0)
    def _():
        sums_ref[...] += contrib

    @pl.when(i == NGRID - 1)
    def _():
        full = sums_ref[...]
        su = full[:, 0:HID]
        cnt0 = full[:, HID:HID + 1]
        m = jnp.dot(su, w2_ref[...], preferred_element_type=jnp.float32)
        m = m + cnt0 * b2_ref[...]
        pooled = m / jnp.maximum(cnt0, 1.0)
        tbl = jnp.dot(pooled, w3p_ref[...],
                      preferred_element_type=jnp.float32)
        HH = HID // 2
        bits = jax.lax.bitcast_convert_type(tbl, jnp.uint32)
        half = jnp.uint32(0x8000)
        hi = (bits[:, 0:HH] + half) & jnp.uint32(0xFFFF0000)
        lo = (bits[:, HH:HID] + half) >> 16
        packed = jax.lax.bitcast_convert_type(hi | lo, jnp.int32)
        rep = jnp.broadcast_to(packed[None], (NREP, NSEG, HH))
        tblp_ref[...] = rep.reshape(NREP * NSEG, HH)


_src_pool = pl.pallas_call(
    _src_pool_body,
    grid=(NGRID,),
    in_specs=[
        pl.BlockSpec((1, 1, BLK), lambda i: (i, 0, 0)),
        pl.BlockSpec((BLK, IN_C), lambda i: (i, 0)),
        pl.BlockSpec((IN_C, HID), lambda i: (0, 0)),
        pl.BlockSpec((1, HID), lambda i: (0, 0)),
        pl.BlockSpec((HID, HID), lambda i: (0, 0)),
        pl.BlockSpec((1, HID), lambda i: (0, 0)),
        pl.BlockSpec((HID, HID), lambda i: (0, 0)),
    ],
    out_specs=[
        pl.BlockSpec((NREP * NSEG, HID // 2), lambda i: (0, 0)),
        pl.BlockSpec((NSEG, UW), lambda i: (0, 0)),
    ],
    out_shape=[
        jax.ShapeDtypeStruct((NREP * NSEG, HID // 2), jnp.int32),
        jax.ShapeDtypeStruct((NSEG, UW), jnp.float32),
    ],
    compiler_params=pltpu.CompilerParams(dimension_semantics=("arbitrary",)),
)


@functools.lru_cache(maxsize=None)
def _make_sc_gather():
    mesh = plsc.VectorSubcoreMesh(core_axis_name="c", subcore_axis_name="s")

    @functools.partial(
        pl.kernel,
        mesh=mesh,
        out_type=[
            jax.ShapeDtypeStruct((NPAD, HID // 2), jnp.int32),
        ],
    scratch_types=[
            pltpu.VMEM((NBLK, SB), jnp.int32),
            pltpu.VMEM((SB, HID // 2), jnp.int32),
            pltpu.VMEM((SB, HID // 2), jnp.int32),
            pltpu.SemaphoreType.DMA,
            pltpu.SemaphoreType.DMA,
            pltpu.SemaphoreType.DMA,
            pltpu.SemaphoreType.DMA,
        ],
    )
    def _sc_gather(tbl_hbm, idx_hbm, g_out,
                   idx_v, rows_a, rows_b, gsem_a, gsem_b, ssem_a, ssem_b):
        c = lax.axis_index("c")
        s = lax.axis_index("s")
        w = s * NC + c
        pltpu.sync_copy(idx_hbm.at[w], idx_v)
        base = w * RPW
        bufs = (rows_a, rows_b)
        gsems = (gsem_a, gsem_b)
        ssems = (ssem_a, ssem_b)
        gath = pltpu.async_copy(tbl_hbm.at[idx_v.at[0]], rows_a, gsem_a)
        scat = [None, None]
        for j in range(NBLK):
            if j + 1 < NBLK:
                b = (j + 1) % 2
                if scat[b] is not None:
                    scat[b].wait()
                    scat[b] = None
                nxt = pltpu.async_copy(tbl_hbm.at[idx_v.at[j + 1]],
                                       bufs[b], gsems[b])
            gath.wait()
            scat[j % 2] = pltpu.async_copy(
                bufs[j % 2], g_out.at[pl.ds(base + j * SB, SB)],
                ssems[j % 2])
            if j + 1 < NBLK:
                gath = nxt
        for a in scat:
            if a is not None:
                a.wait()

    return _sc_gather


def _tgt_mlp_body(g_ref, c_ref, w3c_ref, b3_ref, w4_ref, b4_ref,
                  o_ref):
    cc = c_ref[...]
    w3c = w3c_ref[...]
    HH = HID // 2
    ct = cc[:, 0:1] * w3c[0:1, :] + cc[:, 1:2] * w3c[1:2, :]
    b3 = b3_ref[...]
    bits = jax.lax.bitcast_convert_type(g_ref[...], jnp.uint32)
    g1 = jax.lax.bitcast_convert_type(bits & jnp.uint32(0xFFFF0000),
                                      jnp.float32)
    g2 = jax.lax.bitcast_convert_type(bits << 16, jnp.float32)
    h1 = jax.nn.gelu(g1 + ct[:, 0:HH] + b3[:, 0:HH])
    h2 = jax.nn.gelu(g2 + ct[:, HH:HID] + b3[:, HH:HID])
    w4 = w4_ref[...]
    o_ref[...] = (jnp.dot(h1, w4[0:HH], preferred_element_type=jnp.float32)
                  + jnp.dot(h2, w4[HH:HID],
                            preferred_element_type=jnp.float32)
                  + b4_ref[...])


_tgt_mlp = pl.pallas_call(
    _tgt_mlp_body,
    grid=(NGRID,),
    in_specs=[
        pl.BlockSpec((BLK, HID // 2), lambda i: (i, 0)),
        pl.BlockSpec((BLK, 2), lambda i: (i, 0)),
        pl.BlockSpec((2, HID), lambda i: (0, 0)),
        pl.BlockSpec((1, HID), lambda i: (0, 0)),
        pl.BlockSpec((HID, OUT_C), lambda i: (0, 0)),
        pl.BlockSpec((1, OUT_C), lambda i: (0, 0)),
    ],
    out_specs=pl.BlockSpec((BLK, OUT_C), lambda i: (i, 0)),
    out_shape=jax.ShapeDtypeStruct((N, OUT_C), jnp.float32),
    compiler_params=pltpu.CompilerParams(dimension_semantics=("parallel",)),
)


def _clusters(coords, batch):
    cx = jnp.clip(jnp.floor(coords[:, 0] * NX).astype(jnp.int32), 0, NX - 1)
    cy = jnp.clip(jnp.floor(coords[:, 1] * NY).astype(jnp.int32), 0, NY - 1)
    return batch.astype(jnp.int32) * (NX * NY) + cx * NY + cy


def kernel(x, src_coords, src_batch, tgt_coords, tgt_batch,
           W1, b1, W2, b2, W3, b3, W4, b4):
    src_sub = _clusters(src_coords, src_batch)
    tgt_sub = _clusters(tgt_coords, tgt_batch)
    src_idx3d = jnp.full((NPAD,), NSEG, jnp.int32).at[:N].set(src_sub)
    src_idx3d = src_idx3d.reshape(NGRID, 1, BLK)
    tgt_idx = jnp.zeros((NPAD,), jnp.int32).at[:N].set(tgt_sub)
    tgt_idx3 = tgt_idx.reshape(NW, NBLK, SB)
    rep_of_w = (jnp.arange(NW, dtype=jnp.int32) % NREP) * NSEG
    tgt_idx3 = tgt_idx3 + rep_of_w[:, None, None]

    tblp, _ = _src_pool(src_idx3d, x, W1, b1.reshape(1, HID),
                        W2, b2.reshape(1, HID), W3[2:])
    (g,) = _make_sc_gather()(tblp, tgt_idx3)
    out = _tgt_mlp(g, tgt_coords, W3[:2], b3.reshape(1, HID),
                   W4, b4.reshape(1, OUT_C))
    return out


# speedup vs baseline: 1.0319x; 1.0319x over previous
import functools

import jax
import jax.numpy as jnp
from jax import lax
from jax.experimental import pallas as pl
from jax.experimental.pallas import tpu as pltpu
from jax.experimental.pallas import tpu_sc as plsc

IN_C = 128
OUT_C = 128
HID = 256
NX = 8
NY = 8
NB = 4
NSEG = NB * NX * NY
N = 50000
NPAD = 50176
BLK = 512
NGRID = NPAD // BLK
NC = 2
NS = 16
NW = NC * NS
RPW = NPAD // NW
SB = 112
NBLK = RPW // SB
UW = 384


NREP = NW


def _src_pool_body(ct_ref, bt_ref, x_ref, w1_ref, b1_ref, w2_ref, b2_ref,
                   w3p_ref, tblp_ref, sums_ref):
    i = pl.program_id(0)
    u = jnp.dot(x_ref[...], w1_ref[...], preferred_element_type=jnp.float32)
    u = jax.nn.gelu(u + b1_ref[...])
    u_ext = jnp.concatenate(
        [u, jnp.ones((BLK, UW - HID), jnp.float32)], axis=1)
    rid = i * BLK + jax.lax.broadcasted_iota(jnp.int32, (BLK, 1), 0)
    u_ext = jnp.where(rid < N, u_ext, 0.0)
    ct = ct_ref[...]
    bt = bt_ref[...]
    cx = jnp.clip(jnp.floor(ct[0:1] * NX).astype(jnp.int32), 0, NX - 1)
    cy = jnp.clip(jnp.floor(ct[1:2] * NY).astype(jnp.int32), 0, NY - 1)
    seg = bt * (NX * NY) + cx * NY + cy
    oht = (seg == jax.lax.broadcasted_iota(jnp.int32, (NSEG, BLK), 0)
           ).astype(jnp.float32)
    contrib = jnp.dot(oht, u_ext,
                      preferred_element_type=jnp.float32)

    @pl.when(i == 0)
    def _():
        sums_ref[...] = contrib

    @pl.when(i > 0)
    def _():
        sums_ref[...] += contrib

    @pl.when(i == NGRID - 1)
    def _():
        full = sums_ref[...]
        su = full[:, 0:HID]
        cnt0 = full[:, HID:HID + 1]
        m = jnp.dot(su, w2_ref[...], preferred_element_type=jnp.float32)
        m = m + cnt0 * b2_ref[...]
        pooled = m / jnp.maximum(cnt0, 1.0)
        tbl = jnp.dot(pooled, w3p_ref[...],
                      preferred_element_type=jnp.float32)
        HH = HID // 2
        bits = jax.lax.bitcast_convert_type(tbl, jnp.uint32)
        half = jnp.uint32(0x8000)
        hi = (bits[:, 0:HH] + half) & jnp.uint32(0xFFFF0000)
        lo = (bits[:, HH:HID] + half) >> 16
        packed = jax.lax.bitcast_convert_type(hi | lo, jnp.int32)
        rep = jnp.broadcast_to(packed[None], (NREP, NSEG, HH))
        tblp_ref[...] = rep.reshape(NREP * NSEG, HH)


_src_pool = pl.pallas_call(
    _src_pool_body,
    grid=(NGRID,),
    in_specs=[
        pl.BlockSpec((2, BLK), lambda i: (0, i)),
        pl.BlockSpec((1, BLK), lambda i: (0, i)),
        pl.BlockSpec((BLK, IN_C), lambda i: (i, 0)),
        pl.BlockSpec((IN_C, HID), lambda i: (0, 0)),
        pl.BlockSpec((1, HID), lambda i: (0, 0)),
        pl.BlockSpec((HID, HID), lambda i: (0, 0)),
        pl.BlockSpec((1, HID), lambda i: (0, 0)),
        pl.BlockSpec((HID, HID), lambda i: (0, 0)),
    ],
    out_specs=[
        pl.BlockSpec((NREP * NSEG, HID // 2), lambda i: (0, 0)),
        pl.BlockSpec((NSEG, UW), lambda i: (0, 0)),
    ],
    out_shape=[
        jax.ShapeDtypeStruct((NREP * NSEG, HID // 2), jnp.int32),
        jax.ShapeDtypeStruct((NSEG, UW), jnp.float32),
    ],
    compiler_params=pltpu.CompilerParams(dimension_semantics=("arbitrary",)),
)


@functools.lru_cache(maxsize=None)
def _make_sc_gather():
    mesh = plsc.VectorSubcoreMesh(core_axis_name="c", subcore_axis_name="s")

    @functools.partial(
        pl.kernel,
        mesh=mesh,
        out_type=[
            jax.ShapeDtypeStruct((NPAD, HID // 2), jnp.int32),
        ],
    scratch_types=[
            pltpu.VMEM((NBLK, SB), jnp.int32),
            pltpu.VMEM((SB, HID // 2), jnp.int32),
            pltpu.VMEM((SB, HID // 2), jnp.int32),
            pltpu.SemaphoreType.DMA,
            pltpu.SemaphoreType.DMA,
            pltpu.SemaphoreType.DMA,
            pltpu.SemaphoreType.DMA,
        ],
    )
    def _sc_gather(tbl_hbm, idx_hbm, g_out,
                   idx_v, rows_a, rows_b, gsem_a, gsem_b, ssem_a, ssem_b):
        c = lax.axis_index("c")
        s = lax.axis_index("s")
        w = s * NC + c
        pltpu.sync_copy(idx_hbm.at[w], idx_v)
        idx_v[...] = idx_v[...] + w * NSEG
        base = w * RPW
        bufs = (rows_a, rows_b)
        gsems = (gsem_a, gsem_b)
        ssems = (ssem_a, ssem_b)
        gath = pltpu.async_copy(tbl_hbm.at[idx_v.at[0]], rows_a, gsem_a)
        scat = [None, None]
        for j in range(NBLK):
            if j + 1 < NBLK:
                b = (j + 1) % 2
                if scat[b] is not None:
                    scat[b].wait()
                    scat[b] = None
                nxt = pltpu.async_copy(tbl_hbm.at[idx_v.at[j + 1]],
                                       bufs[b], gsems[b])
            gath.wait()
            scat[j % 2] = pltpu.async_copy(
                bufs[j % 2], g_out.at[pl.ds(base + j * SB, SB)],
                ssems[j % 2])
            if j + 1 < NBLK:
                gath = nxt
        for a in scat:
            if a is not None:
                a.wait()

    return _sc_gather


def _tgt_mlp_body(g_ref, c_ref, w3c_ref, b3_ref, w4_ref, b4_ref,
                  o_ref):
    cc = c_ref[...]
    w3c = w3c_ref[...]
    HH = HID // 2
    ct = cc[:, 0:1] * w3c[0:1, :] + cc[:, 1:2] * w3c[1:2, :]
    b3 = b3_ref[...]
    bits = jax.lax.bitcast_convert_type(g_ref[...], jnp.uint32)
    g1 = jax.lax.bitcast_convert_type(bits & jnp.uint32(0xFFFF0000),
                                      jnp.float32)
    g2 = jax.lax.bitcast_convert_type(bits << 16, jnp.float32)
    h1 = jax.nn.gelu(g1 + ct[:, 0:HH] + b3[:, 0:HH])
    h2 = jax.nn.gelu(g2 + ct[:, HH:HID] + b3[:, HH:HID])
    w4 = w4_ref[...]
    o_ref[...] = (jnp.dot(h1, w4[0:HH], preferred_element_type=jnp.float32)
                  + jnp.dot(h2, w4[HH:HID],
                            preferred_element_type=jnp.float32)
                  + b4_ref[...])


_tgt_mlp = pl.pallas_call(
    _tgt_mlp_body,
    grid=(NGRID,),
    in_specs=[
        pl.BlockSpec((BLK, HID // 2), lambda i: (i, 0)),
        pl.BlockSpec((BLK, 2), lambda i: (i, 0)),
        pl.BlockSpec((2, HID), lambda i: (0, 0)),
        pl.BlockSpec((1, HID), lambda i: (0, 0)),
        pl.BlockSpec((HID, OUT_C), lambda i: (0, 0)),
        pl.BlockSpec((1, OUT_C), lambda i: (0, 0)),
    ],
    out_specs=pl.BlockSpec((BLK, OUT_C), lambda i: (i, 0)),
    out_shape=jax.ShapeDtypeStruct((N, OUT_C), jnp.float32),
    compiler_params=pltpu.CompilerParams(dimension_semantics=("parallel",)),
)


def _clusters(coords, batch):
    cx = jnp.clip(jnp.floor(coords[:, 0] * NX).astype(jnp.int32), 0, NX - 1)
    cy = jnp.clip(jnp.floor(coords[:, 1] * NY).astype(jnp.int32), 0, NY - 1)
    return batch.astype(jnp.int32) * (NX * NY) + cx * NY + cy


def kernel(x, src_coords, src_batch, tgt_coords, tgt_batch,
           W1, b1, W2, b2, W3, b3, W4, b4):
    tgt_sub = _clusters(tgt_coords, tgt_batch)
    tgt_idx = jnp.zeros((NPAD,), jnp.int32).at[:N].set(tgt_sub)
    tgt_idx3 = tgt_idx.reshape(NW, NBLK, SB)

    cT = src_coords.T
    bT = src_batch.astype(jnp.int32).reshape(1, N)
    tblp, _ = _src_pool(cT, bT, x, W1, b1.reshape(1, HID),
                        W2, b2.reshape(1, HID), W3[2:])
    (g,) = _make_sc_gather()(tblp, tgt_idx3)
    out = _tgt_mlp(g, tgt_coords, W3[:2], b3.reshape(1, HID),
                   W4, b4.reshape(1, OUT_C))
    return out


# speedup vs baseline: 1.0334x; 1.0015x over previous
import functools

import jax
import jax.numpy as jnp
from jax import lax
from jax.experimental import pallas as pl
from jax.experimental.pallas import tpu as pltpu
from jax.experimental.pallas import tpu_sc as plsc

IN_C = 128
OUT_C = 128
HID = 256
NX = 8
NY = 8
NB = 4
NSEG = NB * NX * NY
N = 50000
NPAD = 50176
BLK = 512
NGRID = NPAD // BLK
NC = 2
NS = 16
NW = NC * NS
RPW = NPAD // NW
SB = 112
NBLK = RPW // SB
UW = 384


NREP = NW


def _src_pool_body(ct_ref, bt_ref, x_ref, w1_ref, b1_ref, w2_ref, b2_ref,
                   w3p_ref, tblp_ref, sums_ref):
    i = pl.program_id(0)
    u = jnp.dot(x_ref[...], w1_ref[...], preferred_element_type=jnp.float32)
    u = jax.nn.gelu(u + b1_ref[...])
    ct = ct_ref[...]
    bt = bt_ref[...]
    cx = jnp.clip(jnp.floor(ct[0:1] * NX).astype(jnp.int32), 0, NX - 1)
    cy = jnp.clip(jnp.floor(ct[1:2] * NY).astype(jnp.int32), 0, NY - 1)
    seg = bt * (NX * NY) + cx * NY + cy
    lane = jax.lax.broadcasted_iota(jnp.int32, (NSEG, BLK), 1)
    oht = jnp.where(
        (seg == jax.lax.broadcasted_iota(jnp.int32, (NSEG, BLK), 0))
        & (i * BLK + lane < N), 1.0, 0.0)
    contrib = jnp.dot(oht, u,
                      preferred_element_type=jnp.float32)
    cnt = jnp.sum(oht, axis=1, keepdims=True)

    @pl.when(i == 0)
    def _():
        sums_ref[:, 0:HID] = contrib
        sums_ref[:, HID:HID + 1] = cnt

    @pl.when(i > 0)
    def _():
        sums_ref[:, 0:HID] += contrib
        sums_ref[:, HID:HID + 1] += cnt

    @pl.when(i == NGRID - 1)
    def _():
        full = sums_ref[...]
        su = full[:, 0:HID]
        cnt0 = full[:, HID:HID + 1]
        m = jnp.dot(su, w2_ref[...], preferred_element_type=jnp.float32)
        m = m + cnt0 * b2_ref[...]
        pooled = m / jnp.maximum(cnt0, 1.0)
        tbl = jnp.dot(pooled, w3p_ref[...],
                      preferred_element_type=jnp.float32)
        HH = HID // 2
        bits = jax.lax.bitcast_convert_type(tbl, jnp.uint32)
        half = jnp.uint32(0x8000)
        hi = (bits[:, 0:HH] + half) & jnp.uint32(0xFFFF0000)
        lo = (bits[:, HH:HID] + half) >> 16
        packed = jax.lax.bitcast_convert_type(hi | lo, jnp.int32)
        rep = jnp.broadcast_to(packed[None], (NREP, NSEG, HH))
        tblp_ref[...] = rep.reshape(NREP * NSEG, HH)


_src_pool = pl.pallas_call(
    _src_pool_body,
    grid=(NGRID,),
    in_specs=[
        pl.BlockSpec((2, BLK), lambda i: (0, i)),
        pl.BlockSpec((1, BLK), lambda i: (0, i)),
        pl.BlockSpec((BLK, IN_C), lambda i: (i, 0)),
        pl.BlockSpec((IN_C, HID), lambda i: (0, 0)),
        pl.BlockSpec((1, HID), lambda i: (0, 0)),
        pl.BlockSpec((HID, HID), lambda i: (0, 0)),
        pl.BlockSpec((1, HID), lambda i: (0, 0)),
        pl.BlockSpec((HID, HID), lambda i: (0, 0)),
    ],
    out_specs=[
        pl.BlockSpec((NREP * NSEG, HID // 2), lambda i: (0, 0)),
        pl.BlockSpec((NSEG, UW), lambda i: (0, 0)),
    ],
    out_shape=[
        jax.ShapeDtypeStruct((NREP * NSEG, HID // 2), jnp.int32),
        jax.ShapeDtypeStruct((NSEG, UW), jnp.float32),
    ],
    compiler_params=pltpu.CompilerParams(dimension_semantics=("arbitrary",)),
)


@functools.lru_cache(maxsize=None)
def _make_sc_gather():
    mesh = plsc.VectorSubcoreMesh(core_axis_name="c", subcore_axis_name="s")

    @functools.partial(
        pl.kernel,
        mesh=mesh,
        out_type=[
            jax.ShapeDtypeStruct((NPAD, HID // 2), jnp.int32),
        ],
    scratch_types=[
            pltpu.VMEM((NBLK, SB), jnp.int32),
            pltpu.VMEM((SB, HID // 2), jnp.int32),
            pltpu.VMEM((SB, HID // 2), jnp.int32),
            pltpu.SemaphoreType.DMA,
            pltpu.SemaphoreType.DMA,
            pltpu.SemaphoreType.DMA,
            pltpu.SemaphoreType.DMA,
        ],
    )
    def _sc_gather(tbl_hbm, idx_hbm, g_out,
                   idx_v, rows_a, rows_b, gsem_a, gsem_b, ssem_a, ssem_b):
        c = lax.axis_index("c")
        s = lax.axis_index("s")
        w = s * NC + c
        pltpu.sync_copy(idx_hbm.at[w], idx_v)
        idx_v[...] = idx_v[...] + w * NSEG
        base = w * RPW
        bufs = (rows_a, rows_b)
        gsems = (gsem_a, gsem_b)
        ssems = (ssem_a, ssem_b)
        gath = pltpu.async_copy(tbl_hbm.at[idx_v.at[0]], rows_a, gsem_a)
        scat = [None, None]
        for j in range(NBLK):
            if j + 1 < NBLK:
                b = (j + 1) % 2
                if scat[b] is not None:
                    scat[b].wait()
                    scat[b] = None
                nxt = pltpu.async_copy(tbl_hbm.at[idx_v.at[j + 1]],
                                       bufs[b], gsems[b])
            gath.wait()
            scat[j % 2] = pltpu.async_copy(
                bufs[j % 2], g_out.at[pl.ds(base + j * SB, SB)],
                ssems[j % 2])
            if j + 1 < NBLK:
                gath = nxt
        for a in scat:
            if a is not None:
                a.wait()

    return _sc_gather


def _tgt_mlp_body(g_ref, c_ref, w3c_ref, b3_ref, w4_ref, b4_ref,
                  o_ref):
    cc = c_ref[...]
    w3c = w3c_ref[...]
    HH = HID // 2
    ct = cc[:, 0:1] * w3c[0:1, :] + cc[:, 1:2] * w3c[1:2, :]
    b3 = b3_ref[...]
    bits = jax.lax.bitcast_convert_type(g_ref[...], jnp.uint32)
    g1 = jax.lax.bitcast_convert_type(bits & jnp.uint32(0xFFFF0000),
                                      jnp.float32)
    g2 = jax.lax.bitcast_convert_type(bits << 16, jnp.float32)
    h1 = jax.nn.gelu(g1 + ct[:, 0:HH] + b3[:, 0:HH])
    h2 = jax.nn.gelu(g2 + ct[:, HH:HID] + b3[:, HH:HID])
    w4 = w4_ref[...]
    o_ref[...] = (jnp.dot(h1, w4[0:HH], preferred_element_type=jnp.float32)
                  + jnp.dot(h2, w4[HH:HID],
                            preferred_element_type=jnp.float32)
                  + b4_ref[...])


_tgt_mlp = pl.pallas_call(
    _tgt_mlp_body,
    grid=(NGRID,),
    in_specs=[
        pl.BlockSpec((BLK, HID // 2), lambda i: (i, 0)),
        pl.BlockSpec((BLK, 2), lambda i: (i, 0)),
        pl.BlockSpec((2, HID), lambda i: (0, 0)),
        pl.BlockSpec((1, HID), lambda i: (0, 0)),
        pl.BlockSpec((HID, OUT_C), lambda i: (0, 0)),
        pl.BlockSpec((1, OUT_C), lambda i: (0, 0)),
    ],
    out_specs=pl.BlockSpec((BLK, OUT_C), lambda i: (i, 0)),
    out_shape=jax.ShapeDtypeStruct((N, OUT_C), jnp.float32),
    compiler_params=pltpu.CompilerParams(dimension_semantics=("parallel",)),
)


def _clusters(coords, batch):
    cx = jnp.clip(jnp.floor(coords[:, 0] * NX).astype(jnp.int32), 0, NX - 1)
    cy = jnp.clip(jnp.floor(coords[:, 1] * NY).astype(jnp.int32), 0, NY - 1)
    return batch.astype(jnp.int32) * (NX * NY) + cx * NY + cy


def kernel(x, src_coords, src_batch, tgt_coords, tgt_batch,
           W1, b1, W2, b2, W3, b3, W4, b4):
    tgt_sub = _clusters(tgt_coords, tgt_batch)
    tgt_idx = jnp.zeros((NPAD,), jnp.int32).at[:N].set(tgt_sub)
    tgt_idx3 = tgt_idx.reshape(NW, NBLK, SB)

    cT = src_coords.T
    bT = src_batch.astype(jnp.int32).reshape(1, N)
    tblp, _ = _src_pool(cT, bT, x, W1, b1.reshape(1, HID),
                        W2, b2.reshape(1, HID), W3[2:])
    (g,) = _make_sc_gather()(tblp, tgt_idx3)
    out = _tgt_mlp(g, tgt_coords, W3[:2], b3.reshape(1, HID),
                   W4, b4.reshape(1, OUT_C))
    return out
